# Initial kernel scaffold; baseline (speedup 1.0000x reference)
import jax
import jax.numpy as jnp
from jax.experimental import pallas as pl


def kernel(users, items, categories, subcategories, entities, user_table, news_table, cat_table, subcat_table, entity_table, bert_table, W_bert, b_bert, W_content, b_content):
    raise NotImplementedError("write your pallas kernel here")


# speedup vs baseline: 2.3371x; 2.3371x over previous
import functools

import jax
import jax.numpy as jnp
from jax import lax
from jax.experimental import pallas as pl
from jax.experimental.pallas import tpu as pltpu
from jax.experimental.pallas import tpu_sc as plsc

B = 16384
EMB = 50
BERT_DIM = 768
NW = 32
BPW = B // NW
CHUNK = 128
NCH = BPW // CHUNK


def _sc_gather_bert(items, bert_table):
    mesh = plsc.VectorSubcoreMesh(core_axis_name="c", subcore_axis_name="s")
    out_type = jax.ShapeDtypeStruct((B, BERT_DIM), jnp.float32)

    @functools.partial(
        pl.kernel, mesh=mesh, out_type=out_type,
        scratch_types=[
            pltpu.VMEM((BPW,), jnp.int32),
            pltpu.VMEM((CHUNK, BERT_DIM), jnp.float32),
            pltpu.SemaphoreType.DMA,
        ],
    )
    def k(items_h, bert_t, out_bert, idx_v, row768_v, sem):
        wid = lax.axis_index("s") * 2 + lax.axis_index("c")
        base = wid * BPW
        pltpu.sync_copy(items_h.at[pl.ds(base, BPW)], idx_v)
        for c in range(NCH):
            pltpu.async_copy(
                bert_t.at[idx_v.at[pl.ds(c * CHUNK, CHUNK)]], row768_v, sem
            ).wait()
            pltpu.sync_copy(row768_v, out_bert.at[pl.ds(base + c * CHUNK, CHUNK)])

    return k(items, bert_table)


BLK = 512


def _tc_body(u_ref, n_ref, c_ref, s_ref, e_ref, bt_ref,
             wb_ref, bb_ref, wc_ref, bc_ref, o_ref):
    bert = jax.nn.sigmoid(
        jnp.dot(bt_ref[...], wb_ref[...], preferred_element_type=jnp.float32)
        + bb_ref[...])
    wc = wc_ref[...]
    z = (jnp.dot(n_ref[...], wc[0:EMB], preferred_element_type=jnp.float32)
         + jnp.dot(bert, wc[EMB:2 * EMB], preferred_element_type=jnp.float32)
         + jnp.dot(c_ref[...], wc[2 * EMB:3 * EMB],
                   preferred_element_type=jnp.float32)
         + jnp.dot(s_ref[...], wc[3 * EMB:4 * EMB],
                   preferred_element_type=jnp.float32)
         + jnp.dot(e_ref[...], wc[4 * EMB:5 * EMB],
                   preferred_element_type=jnp.float32)
         + bc_ref[...])
    nc = jax.nn.sigmoid(z)
    o_ref[...] = jax.nn.sigmoid(jnp.sum(u_ref[...] * nc, axis=1))


def _tc_compute(user50, news50, cat50, subcat50, ent50, bert768,
                W_bert, b_bert, W_content, b_content):
    grid = B // BLK
    row_spec = pl.BlockSpec((BLK, EMB), lambda i: (i, 0))
    bert_spec = pl.BlockSpec((BLK, BERT_DIM), lambda i: (i, 0))
    full = lambda shape: pl.BlockSpec(shape, lambda i: (0,) * len(shape))
    return pl.pallas_call(
        _tc_body,
        grid=(grid,),
        in_specs=[row_spec, row_spec, row_spec, row_spec, row_spec, bert_spec,
                  full((BERT_DIM, EMB)), full((EMB,)),
                  full((5 * EMB, EMB)), full((EMB,))],
        out_specs=pl.BlockSpec((BLK,), lambda i: (i,)),
        out_shape=jax.ShapeDtypeStruct((B,), jnp.float32),
    )(user50, news50, cat50, subcat50, ent50, bert768,
      W_bert, b_bert, W_content, b_content)


def kernel(users, items, categories, subcategories, entities,
           user_table, news_table, cat_table, subcat_table, entity_table,
           bert_table, W_bert, b_bert, W_content, b_content):
    ent0 = entities[:, 0]
    bert768 = _sc_gather_bert(items, bert_table)
    user50 = jnp.take(user_table, users, axis=0)
    news50 = jnp.take(news_table, items, axis=0)
    cat50 = jnp.take(cat_table, categories, axis=0)
    subcat50 = jnp.take(subcat_table, subcategories, axis=0)
    ent50 = jnp.take(entity_table, ent0, axis=0)
    return _tc_compute(user50, news50, cat50, subcat50, ent50, bert768,
                       W_bert, b_bert, W_content, b_content)


# speedup vs baseline: 2.3397x; 1.0011x over previous
import functools

import jax
import jax.numpy as jnp
from jax import lax
from jax.experimental import pallas as pl
from jax.experimental.pallas import tpu as pltpu
from jax.experimental.pallas import tpu_sc as plsc

B = 16384
EMB = 50
BERT_DIM = 768
NW = 32
BPW = B // NW
CHUNK = 128
NCH = BPW // CHUNK


def _sc_gather_small(users, items, categories, subcategories, ent0,
                     user_table, news_table, cat_table, subcat_table,
                     entity_table):
    mesh = plsc.VectorSubcoreMesh(core_axis_name="c", subcore_axis_name="s")
    out_type = tuple(jax.ShapeDtypeStruct((B, EMB), jnp.float32)
                     for _ in range(5))

    @functools.partial(
        pl.kernel, mesh=mesh, out_type=out_type,
        scratch_types=[
            pltpu.VMEM((BPW,), jnp.int32),
            pltpu.VMEM((CHUNK, EMB), jnp.float32),
            pltpu.SemaphoreType.DMA,
        ],
        compiler_params=pltpu.CompilerParams(use_tc_tiling_on_sc=False),
    )
    def k(users_h, items_h, cats_h, subcats_h, ent_h,
          user_t, news_t, cat_t, subcat_t, ent_t,
          out_user, out_news, out_cat, out_subcat, out_ent,
          idx_v, row50_v, sem):
        wid = lax.axis_index("s") * 2 + lax.axis_index("c")
        base = wid * BPW
        small = ((users_h, user_t, out_user),
                 (items_h, news_t, out_news),
                 (cats_h, cat_t, out_cat),
                 (subcats_h, subcat_t, out_subcat),
                 (ent_h, ent_t, out_ent))
        for idx_h, tab, out in small:
            pltpu.sync_copy(idx_h.at[pl.ds(base, BPW)], idx_v)
            for c in range(NCH):
                pltpu.async_copy(
                    tab.at[idx_v.at[pl.ds(c * CHUNK, CHUNK)]], row50_v, sem
                ).wait()
                pltpu.sync_copy(row50_v, out.at[pl.ds(base + c * CHUNK, CHUNK)])

    return k(users, items, categories, subcategories, ent0,
             user_table, news_table, cat_table, subcat_table, entity_table)


def _sc_gather_bert(items, bert_table):
    mesh = plsc.VectorSubcoreMesh(core_axis_name="c", subcore_axis_name="s")
    out_type = jax.ShapeDtypeStruct((B, BERT_DIM), jnp.float32)

    @functools.partial(
        pl.kernel, mesh=mesh, out_type=out_type,
        scratch_types=[
            pltpu.VMEM((BPW,), jnp.int32),
            pltpu.VMEM((CHUNK, BERT_DIM), jnp.float32),
            pltpu.SemaphoreType.DMA,
        ],
    )
    def k(items_h, bert_t, out_bert, idx_v, row768_v, sem):
        wid = lax.axis_index("s") * 2 + lax.axis_index("c")
        base = wid * BPW
        pltpu.sync_copy(items_h.at[pl.ds(base, BPW)], idx_v)
        for c in range(NCH):
            pltpu.async_copy(
                bert_t.at[idx_v.at[pl.ds(c * CHUNK, CHUNK)]], row768_v, sem
            ).wait()
            pltpu.sync_copy(row768_v, out_bert.at[pl.ds(base + c * CHUNK, CHUNK)])

    return k(items, bert_table)


BLK = 512


def _tc_body(u_ref, n_ref, c_ref, s_ref, e_ref, bt_ref,
             wb_ref, bb_ref, wc_ref, bc_ref, o_ref):
    bert = jax.nn.sigmoid(
        jnp.dot(bt_ref[...], wb_ref[...], preferred_element_type=jnp.float32)
        + bb_ref[...])
    wc = wc_ref[...]
    z = (jnp.dot(n_ref[...], wc[0:EMB], preferred_element_type=jnp.float32)
         + jnp.dot(bert, wc[EMB:2 * EMB], preferred_element_type=jnp.float32)
         + jnp.dot(c_ref[...], wc[2 * EMB:3 * EMB],
                   preferred_element_type=jnp.float32)
         + jnp.dot(s_ref[...], wc[3 * EMB:4 * EMB],
                   preferred_element_type=jnp.float32)
         + jnp.dot(e_ref[...], wc[4 * EMB:5 * EMB],
                   preferred_element_type=jnp.float32)
         + bc_ref[...])
    nc = jax.nn.sigmoid(z)
    o_ref[...] = jax.nn.sigmoid(jnp.sum(u_ref[...] * nc, axis=1))


def _tc_compute(user50, news50, cat50, subcat50, ent50, bert768,
                W_bert, b_bert, W_content, b_content):
    grid = B // BLK
    row_spec = pl.BlockSpec((BLK, EMB), lambda i: (i, 0))
    bert_spec = pl.BlockSpec((BLK, BERT_DIM), lambda i: (i, 0))
    full = lambda shape: pl.BlockSpec(shape, lambda i: (0,) * len(shape))
    return pl.pallas_call(
        _tc_body,
        grid=(grid,),
        in_specs=[row_spec, row_spec, row_spec, row_spec, row_spec, bert_spec,
                  full((BERT_DIM, EMB)), full((EMB,)),
                  full((5 * EMB, EMB)), full((EMB,))],
        out_specs=pl.BlockSpec((BLK,), lambda i: (i,)),
        out_shape=jax.ShapeDtypeStruct((B,), jnp.float32),
    )(user50, news50, cat50, subcat50, ent50, bert768,
      W_bert, b_bert, W_content, b_content)


def kernel(users, items, categories, subcategories, entities,
           user_table, news_table, cat_table, subcat_table, entity_table,
           bert_table, W_bert, b_bert, W_content, b_content):
    ent0 = entities[:, 0]
    bert768 = _sc_gather_bert(items, bert_table)
    user50 = jnp.take(user_table, users, axis=0)
    news50 = jnp.take(news_table, items, axis=0)
    cat50 = jnp.take(cat_table, categories, axis=0)
    subcat50 = jnp.take(subcat_table, subcategories, axis=0)
    ent50 = jnp.take(entity_table, ent0, axis=0)
    return _tc_compute(user50, news50, cat50, subcat50, ent50, bert768,
                       W_bert, b_bert, W_content, b_content)
